# baseline (speedup 1.0000x reference)
import functools

import jax
import jax.numpy as jnp
from jax import lax
from jax.experimental import pallas as pl
from jax.experimental.pallas import tpu as pltpu
from jax.experimental.pallas import tpu_sc as plsc

NUM_TOKENS = 1000000
DIM = 64
BATCH = 4096
SEQ = 200

N = BATCH * SEQ
NW = 32
PER_W = N // NW
CHUNK = 1024
STREAM = 128
NSTREAM = CHUNK // STREAM
NCHUNK = PER_W // CHUNK


def _rsqrt(x):
    i = lax.bitcast_convert_type(x, jnp.int32)
    i = jnp.int32(0x5F3759DF) - lax.shift_right_logical(i, 1)
    y = lax.bitcast_convert_type(i, jnp.float32)
    for _ in range(3):
        y = y * (1.5 - 0.5 * x * y * y)
    return y


def _body(x_hbm, emb_hbm, out_hbm, idx_v, rows_v, p_buf, s_buf, sem):
    wid = lax.axis_index("s") * 2 + lax.axis_index("c")
    base = wid * PER_W

    @pl.loop(0, NCHUNK)
    def _chunk(g):
        off = base + g * CHUNK
        pltpu.sync_copy(x_hbm.at[wid * NCHUNK + g], idx_v)
        copies = []
        for j in range(NSTREAM):
            copies.append(pltpu.async_copy(
                emb_hbm.at[idx_v.at[j]],
                rows_v.at[pl.ds(j * STREAM, STREAM)], sem))
        for c in copies:
            c.wait()

        lanes = lax.iota(jnp.int32, 16)

        @pl.loop(0, CHUNK // 16)
        def _grp(t):
            r0 = t * 16
            for r in range(16):
                v0 = rows_v[r0 + r, pl.ds(0, 16)]
                v1 = rows_v[r0 + r, pl.ds(16, 16)]
                v2 = rows_v[r0 + r, pl.ds(32, 16)]
                v3 = rows_v[r0 + r, pl.ds(48, 16)]
                p_buf[r, :] = v0 * v0 + v1 * v1 + v2 * v2 + v3 * v3
            acc = plsc.load_gather(p_buf, [lanes, jnp.zeros((16,), jnp.int32)])
            for l in range(1, 16):
                acc = acc + plsc.load_gather(
                    p_buf, [lanes, jnp.full((16,), l, jnp.int32)])
            s = _rsqrt(jnp.maximum(acc, 1e-24))
            for r in range(16):
                sr = jnp.broadcast_to(s[r], (16,))
                for k in range(4):
                    rows_v[r0 + r, pl.ds(k * 16, 16)] = (
                        rows_v[r0 + r, pl.ds(k * 16, 16)] * sr)

        pltpu.sync_copy(rows_v, out_hbm.at[pl.ds(off, CHUNK)])


@functools.partial(
    pl.kernel,
    out_type=jax.ShapeDtypeStruct((N, DIM), jnp.float32),
    mesh=plsc.VectorSubcoreMesh(core_axis_name="c", subcore_axis_name="s"),
    compiler_params=pltpu.CompilerParams(
        needs_layout_passes=False, use_tc_tiling_on_sc=False),
    scratch_types=[
        pltpu.VMEM((NSTREAM, STREAM), jnp.int32),
        pltpu.VMEM((CHUNK, DIM), jnp.float32),
        pltpu.VMEM((16, 16), jnp.float32),
        pltpu.VMEM((16,), jnp.float32),
        pltpu.SemaphoreType.DMA,
    ],
)
def _embed_norm(x_hbm, emb_hbm, out_hbm, idx_v, rows_v, p_buf, s_buf, sem):
    _body(x_hbm, emb_hbm, out_hbm, idx_v, rows_v, p_buf, s_buf, sem)


def kernel(x, embed):
    x2d = jnp.reshape(x.astype(jnp.int32), (N // CHUNK, NSTREAM, STREAM))
    out = _embed_norm(x2d, embed)
    return jnp.reshape(out, (BATCH, SEQ, DIM))
